# baseline (speedup 1.0000x reference)
import jax
import jax.numpy as jnp
from jax.experimental import pallas as pl
from jax.experimental.pallas import tpu as pltpu


def kernel(x, idxs):
    B, C, T, H, W = x.shape
    BC = B * C
    xr = x.reshape(BC, T, H, W)
    idxs32 = idxs.astype(jnp.int32)

    BC_BLK = 4
    grid = (BC // BC_BLK,)

    def body(idx_ref, x_ref, o_ref):
        for t in range(T):
            o_ref[:, t] = x_ref[:, idx_ref[t]]

    out = pl.pallas_call(
        body,
        grid_spec=pltpu.PrefetchScalarGridSpec(
            num_scalar_prefetch=1,
            grid=grid,
            in_specs=[
                pl.BlockSpec((BC_BLK, T, H, W), lambda i, idx_ref: (i, 0, 0, 0))
            ],
            out_specs=pl.BlockSpec(
                (BC_BLK, T, H, W), lambda i, idx_ref: (i, 0, 0, 0)
            ),
        ),
        out_shape=jax.ShapeDtypeStruct((BC, T, H, W), x.dtype),
    )(idxs32, xr)
    return out.reshape(B, C, T, H, W)


# speedup vs baseline: 2.7897x; 2.7897x over previous
import jax
import jax.numpy as jnp
from jax.experimental import pallas as pl
from jax.experimental.pallas import tpu as pltpu


def _copy_body(idx_ref, x_ref, o_ref):
    o_ref[...] = x_ref[...]


def kernel(x, idxs):
    B, C, T, H, W = x.shape
    xt = jnp.transpose(x, (0, 2, 3, 4, 1))
    idxs32 = idxs.astype(jnp.int32)

    out_t = pl.pallas_call(
        _copy_body,
        grid_spec=pltpu.PrefetchScalarGridSpec(
            num_scalar_prefetch=1,
            grid=(B, T),
            in_specs=[
                pl.BlockSpec(
                    (1, 1, H, W, C),
                    lambda b, t, idx_ref: (b, idx_ref[t], 0, 0, 0),
                )
            ],
            out_specs=pl.BlockSpec(
                (1, 1, H, W, C),
                lambda b, t, idx_ref: (b, t, 0, 0, 0),
            ),
        ),
        out_shape=jax.ShapeDtypeStruct((B, T, H, W, C), x.dtype),
    )(idxs32, xt)
    return jnp.transpose(out_t, (0, 4, 1, 2, 3))


# speedup vs baseline: 3.2741x; 1.1736x over previous
import jax
import jax.numpy as jnp
from jax.experimental import pallas as pl
from jax.experimental.pallas import tpu as pltpu


def _copy_body(idx_ref, x_ref, o_ref):
    o_ref[...] = x_ref[...]


def kernel(x, idxs):
    B, C, T, H, W = x.shape
    xt = jnp.transpose(x, (0, 2, 3, 4, 1))
    idxs32 = idxs.astype(jnp.int32)

    out_t = pl.pallas_call(
        _copy_body,
        grid_spec=pltpu.PrefetchScalarGridSpec(
            num_scalar_prefetch=1,
            grid=(B // 2, T),
            in_specs=[
                pl.BlockSpec(
                    (2, 1, H, W, C),
                    lambda b, t, idx_ref: (b, idx_ref[t], 0, 0, 0),
                )
            ],
            out_specs=pl.BlockSpec(
                (2, 1, H, W, C),
                lambda b, t, idx_ref: (b, t, 0, 0, 0),
            ),
        ),
        out_shape=jax.ShapeDtypeStruct((B, T, H, W, C), x.dtype),
    )(idxs32, xt)
    return jnp.transpose(out_t, (0, 4, 1, 2, 3))


# speedup vs baseline: 3.3599x; 1.0262x over previous
import jax
import jax.numpy as jnp
from jax.experimental import pallas as pl
from jax.experimental.pallas import tpu as pltpu


def _copy_body(idx_ref, x_ref, o_ref):
    o_ref[...] = x_ref[...]


def kernel(x, idxs):
    B, C, T, H, W = x.shape
    xt = jnp.transpose(x, (0, 2, 3, 4, 1))
    idxs32 = idxs.astype(jnp.int32)

    out_t = pl.pallas_call(
        _copy_body,
        grid_spec=pltpu.PrefetchScalarGridSpec(
            num_scalar_prefetch=1,
            grid=(B // 4, T),
            in_specs=[
                pl.BlockSpec(
                    (4, 1, H, W, C),
                    lambda b, t, idx_ref: (b, idx_ref[t], 0, 0, 0),
                )
            ],
            out_specs=pl.BlockSpec(
                (4, 1, H, W, C),
                lambda b, t, idx_ref: (b, t, 0, 0, 0),
            ),
        ),
        out_shape=jax.ShapeDtypeStruct((B, T, H, W, C), x.dtype),
    )(idxs32, xt)
    return jnp.transpose(out_t, (0, 4, 1, 2, 3))
